# Initial kernel scaffold; baseline (speedup 1.0000x reference)
import jax
import jax.numpy as jnp
from jax.experimental import pallas as pl


def kernel(x, edge_index, edge_attr, W1, b1, We, be, Wc1, bc1, Wc2, bc2):
    raise NotImplementedError("write your pallas kernel here")


# speedup vs baseline: 26.2001x; 26.2001x over previous
import functools

import jax
import jax.numpy as jnp
from jax import lax
from jax.experimental import pallas as pl
from jax.experimental.pallas import tpu as pltpu
from jax.experimental.pallas import tpu_sc as plsc

N = 10000
E = 320000
F = 128
FH = F // 2
NC = 2
NS = 16
B = 125
EPS = E // NS
KB = EPS // B
RPS = 624
TAIL = N - NS * RPS
DW = 16

_mesh = plsc.VectorSubcoreMesh(
    core_axis_name="c", subcore_axis_name="s", num_cores=NC, num_subcores=NS)


def _rows_copy(src_ref, dst_ref, sid):
    pltpu.sync_copy(src_ref.at[pl.ds(sid * RPS, RPS)],
                    dst_ref.at[pl.ds(sid * RPS, RPS)])

    @pl.when(sid == 0)
    def _():
        pltpu.sync_copy(src_ref.at[pl.ds(NS * RPS, TAIL)],
                        dst_ref.at[pl.ds(NS * RPS, TAIL)])


@functools.partial(
    pl.kernel,
    out_type=jax.ShapeDtypeStruct((NC, N, DW), jnp.float32),
    mesh=_mesh,
    compiler_params=pltpu.CompilerParams(use_tc_tiling_on_sc=False),
    scratch_types=[
        pltpu.VMEM((KB // 2, B), jnp.int32),
        pltpu.VMEM((B, DW), jnp.float32),
        pltpu.VMEM_SHARED((N, DW), jnp.float32),
    ],
)
def _deg_sc(dst_hbm, ones_hbm, zeros_hbm, out_hbm, dst_v, ones_v, acc):
    cid = lax.axis_index("c")
    sid = lax.axis_index("s")
    wid = cid * NS + sid
    _rows_copy(zeros_hbm, acc, sid)
    pltpu.sync_copy(dst_hbm.at[pl.ds(wid * (KB // 2), KB // 2)], dst_v)
    pltpu.sync_copy(ones_hbm, ones_v)
    plsc.subcore_barrier()

    def body(j, _):
        pltpu.sync_copy(ones_v, acc.at[dst_v.at[j]], add=True)
        return 0

    lax.fori_loop(0, KB // 2, body, 0)
    plsc.subcore_barrier()
    _rows_copy(acc, out_hbm.at[cid], sid)


@functools.partial(
    pl.kernel,
    out_type=jax.ShapeDtypeStruct((NC, N, FH), jnp.float32),
    mesh=_mesh,
    compiler_params=pltpu.CompilerParams(use_tc_tiling_on_sc=False),
    scratch_types=[
        pltpu.VMEM((KB, B), jnp.int32),
        pltpu.VMEM((KB, B), jnp.int32),
        pltpu.VMEM((B, FH), jnp.float32),
        pltpu.VMEM((B, FH), jnp.float32),
        pltpu.VMEM_SHARED((N, FH), jnp.float32),
        pltpu.SemaphoreType.DMA,
        pltpu.SemaphoreType.DMA,
    ],
)
def _spmm_sc(g_hbm, src_hbm, dst_hbm, out_hbm,
             src_v, dst_v, buf0, buf1, acc, sem0, sem1):
    cid = lax.axis_index("c")
    sid = lax.axis_index("s")
    gh = g_hbm.at[cid]
    _rows_copy(gh, acc, sid)
    pltpu.sync_copy(src_hbm.at[pl.ds(sid * KB, KB)], src_v)
    pltpu.sync_copy(dst_hbm.at[pl.ds(sid * KB, KB)], dst_v)
    plsc.subcore_barrier()

    pltpu.async_copy(gh.at[src_v.at[0]], buf0, sem0)
    pltpu.async_copy(gh.at[src_v.at[1]], buf1, sem1)

    def body(i, _):
        j0 = 2 * i
        pltpu.make_async_copy(gh.at[src_v.at[j0]], buf0, sem0).wait()
        pltpu.sync_copy(buf0, acc.at[dst_v.at[j0]], add=True)

        @pl.when(j0 + 2 < KB)
        def _():
            pltpu.async_copy(gh.at[src_v.at[j0 + 2]], buf0, sem0)

        pltpu.make_async_copy(gh.at[src_v.at[j0 + 1]], buf1, sem1).wait()
        pltpu.sync_copy(buf1, acc.at[dst_v.at[j0 + 1]], add=True)

        @pl.when(j0 + 3 < KB)
        def _():
            pltpu.async_copy(gh.at[src_v.at[j0 + 3]], buf1, sem1)

        return 0

    lax.fori_loop(0, KB // 2, body, 0)
    plsc.subcore_barrier()
    _rows_copy(acc, out_hbm.at[cid], sid)


_R = 1000


def _dinv_of(degp_blk):
    deg = degp_blk[0] + degp_blk[1]
    return lax.rsqrt(deg[:, :1] + 1.0)


def _halves(rows):
    return jnp.stack([rows[:, :FH], rows[:, FH:]], axis=0)


def _cat(sp_ref):
    return jnp.concatenate([sp_ref[0], sp_ref[1]], axis=1)


def _tc_in_body(x_ref, w1_ref, b1_ref, wc1_ref, degp_ref, g1_ref):
    h0 = jnp.dot(x_ref[...], w1_ref[...],
                 preferred_element_type=jnp.float32) + b1_ref[...]
    g1_ref[...] = _halves(_dinv_of(degp_ref) * jnp.dot(
        h0, wc1_ref[...], preferred_element_type=jnp.float32))


def _tc_mid_body(sp_ref, degp_ref, b_ref, w_ref, out_ref):
    dinv = _dinv_of(degp_ref)
    h = jax.nn.relu(dinv * _cat(sp_ref) + b_ref[...])
    out_ref[...] = _halves(
        dinv * jnp.dot(h, w_ref[...], preferred_element_type=jnp.float32))


def _tc_out_body(sp_ref, degp_ref, b_ref, out_ref):
    out_ref[...] = jax.nn.relu(
        _dinv_of(degp_ref) * _cat(sp_ref) + b_ref[...])


_row_spec = pl.BlockSpec((_R, F), lambda i: (i, 0))
_mat_spec = pl.BlockSpec((F, F), lambda i: (0, 0))
_bias_spec = pl.BlockSpec((1, F), lambda i: (0, 0))
_degp_spec = pl.BlockSpec((NC, _R, DW), lambda i: (0, i, 0))
_half_spec = pl.BlockSpec((NC, _R, FH), lambda i: (0, i, 0))
_grid = (N // _R,)
_half_out = jax.ShapeDtypeStruct((NC, N, FH), jnp.float32)

_tc_in = pl.pallas_call(
    _tc_in_body, grid=_grid,
    in_specs=[_row_spec, _mat_spec, _bias_spec, _mat_spec, _degp_spec],
    out_specs=_half_spec, out_shape=_half_out)

_tc_mid = pl.pallas_call(
    _tc_mid_body, grid=_grid,
    in_specs=[_half_spec, _degp_spec, _bias_spec, _mat_spec],
    out_specs=_half_spec, out_shape=_half_out)

_tc_out = pl.pallas_call(
    _tc_out_body, grid=_grid,
    in_specs=[_half_spec, _degp_spec, _bias_spec],
    out_specs=_row_spec, out_shape=jax.ShapeDtypeStruct((N, F), jnp.float32))


def kernel(x, edge_index, edge_attr, W1, b1, We, be, Wc1, bc1, Wc2, bc2):
    del edge_attr, We, be
    src2 = edge_index[0].reshape(NS * KB, B)
    dst2 = edge_index[1].reshape(NS * KB, B)
    ones = jnp.ones((B, DW), jnp.float32)
    zeros = jnp.zeros((N, DW), jnp.float32)

    degp = _deg_sc(dst2, ones, zeros)
    g1 = _tc_in(x, W1, b1.reshape(1, F), Wc1, degp)
    s1 = _spmm_sc(g1, src2, dst2)
    g2 = _tc_mid(s1, degp, bc1.reshape(1, F), Wc2)
    s2 = _spmm_sc(g2, src2, dst2)
    return _tc_out(s2, degp, bc2.reshape(1, F))


# speedup vs baseline: 27.1650x; 1.0368x over previous
import functools

import jax
import jax.numpy as jnp
from jax import lax
from jax.experimental import pallas as pl
from jax.experimental.pallas import tpu as pltpu
from jax.experimental.pallas import tpu_sc as plsc

N = 10000
E = 320000
F = 128
FH = F // 2
NC = 2
NS = 16
B = 125
EPS = E // NS
KB = EPS // B
NBUF = 4
RPS = 624
TAIL = N - NS * RPS
DW = 16

_mesh = plsc.VectorSubcoreMesh(
    core_axis_name="c", subcore_axis_name="s", num_cores=NC, num_subcores=NS)


def _rows_copy(src_ref, dst_ref, sid):
    pltpu.sync_copy(src_ref.at[pl.ds(sid * RPS, RPS)],
                    dst_ref.at[pl.ds(sid * RPS, RPS)])

    @pl.when(sid == 0)
    def _():
        pltpu.sync_copy(src_ref.at[pl.ds(NS * RPS, TAIL)],
                        dst_ref.at[pl.ds(NS * RPS, TAIL)])


@functools.partial(
    pl.kernel,
    out_type=jax.ShapeDtypeStruct((NC, N, DW), jnp.float32),
    mesh=_mesh,
    compiler_params=pltpu.CompilerParams(use_tc_tiling_on_sc=False),
    scratch_types=[
        pltpu.VMEM((KB // 2, B), jnp.int32),
        pltpu.VMEM((B, DW), jnp.float32),
        pltpu.VMEM_SHARED((N, DW), jnp.float32),
    ],
)
def _deg_sc(dst_hbm, ones_hbm, zeros_hbm, out_hbm, dst_v, ones_v, acc):
    cid = lax.axis_index("c")
    sid = lax.axis_index("s")
    wid = cid * NS + sid
    _rows_copy(zeros_hbm, acc, sid)
    pltpu.sync_copy(dst_hbm.at[pl.ds(wid * (KB // 2), KB // 2)], dst_v)
    pltpu.sync_copy(ones_hbm, ones_v)
    plsc.subcore_barrier()

    def body(j, _):
        pltpu.sync_copy(ones_v, acc.at[dst_v.at[j]], add=True)
        return 0

    lax.fori_loop(0, KB // 2, body, 0)
    plsc.subcore_barrier()
    _rows_copy(acc, out_hbm.at[cid], sid)


@functools.partial(
    pl.kernel,
    out_type=jax.ShapeDtypeStruct((NC, N, FH), jnp.float32),
    mesh=_mesh,
    compiler_params=pltpu.CompilerParams(use_tc_tiling_on_sc=False),
    scratch_types=[
        pltpu.VMEM((KB, B), jnp.int32),
        pltpu.VMEM((KB, B), jnp.int32),
        pltpu.VMEM((NBUF, B, FH), jnp.float32),
        pltpu.VMEM_SHARED((N, FH), jnp.float32),
        pltpu.SemaphoreType.DMA((NBUF,)),
        pltpu.SemaphoreType.DMA((NBUF,)),
    ],
)
def _spmm_sc(g_hbm, src_hbm, dst_hbm, out_hbm,
             src_v, dst_v, bufs, acc, gsem, ssem):
    cid = lax.axis_index("c")
    sid = lax.axis_index("s")
    gh = g_hbm.at[cid]
    _rows_copy(gh, acc, sid)
    pltpu.sync_copy(src_hbm.at[pl.ds(sid * KB, KB)], src_v)
    pltpu.sync_copy(dst_hbm.at[pl.ds(sid * KB, KB)], dst_v)
    plsc.subcore_barrier()

    for b in range(NBUF):
        pltpu.async_copy(gh.at[src_v.at[b]], bufs.at[b], gsem.at[b])

    def body(i, _):
        for b in range(NBUF):
            j = i * NBUF + b
            pltpu.make_async_copy(
                gh.at[src_v.at[j]], bufs.at[b], gsem.at[b]).wait()
            pltpu.async_copy(bufs.at[b], acc.at[dst_v.at[j]], ssem.at[b],
                             add=True)
            c = (b + NBUF // 2) % NBUF
            k = j - NBUF // 2

            @pl.when(jnp.logical_and(k >= 0, k + NBUF < KB))
            def _():
                pltpu.make_async_copy(
                    bufs.at[c], acc.at[dst_v.at[k]], ssem.at[c]).wait()
                pltpu.async_copy(
                    gh.at[src_v.at[k + NBUF]], bufs.at[c], gsem.at[c])

        return 0

    lax.fori_loop(0, KB // NBUF, body, 0)
    for b in range(NBUF):
        pltpu.make_async_copy(
            bufs.at[b], acc.at[dst_v.at[KB - NBUF + b]], ssem.at[b]).wait()

    plsc.subcore_barrier()
    _rows_copy(acc, out_hbm.at[cid], sid)


_R = 1000


def _dinv_of(degp_blk):
    deg = degp_blk[0] + degp_blk[1]
    return lax.rsqrt(deg[:, :1] + 1.0)


def _halves(rows):
    return jnp.stack([rows[:, :FH], rows[:, FH:]], axis=0)


def _cat(sp_ref):
    return jnp.concatenate([sp_ref[0], sp_ref[1]], axis=1)


def _tc_in_body(x_ref, w1_ref, b1_ref, wc1_ref, degp_ref, g1_ref):
    h0 = jnp.dot(x_ref[...], w1_ref[...],
                 preferred_element_type=jnp.float32) + b1_ref[...]
    g1_ref[...] = _halves(_dinv_of(degp_ref) * jnp.dot(
        h0, wc1_ref[...], preferred_element_type=jnp.float32))


def _tc_mid_body(sp_ref, degp_ref, b_ref, w_ref, out_ref):
    dinv = _dinv_of(degp_ref)
    h = jax.nn.relu(dinv * _cat(sp_ref) + b_ref[...])
    out_ref[...] = _halves(
        dinv * jnp.dot(h, w_ref[...], preferred_element_type=jnp.float32))


def _tc_out_body(sp_ref, degp_ref, b_ref, out_ref):
    out_ref[...] = jax.nn.relu(
        _dinv_of(degp_ref) * _cat(sp_ref) + b_ref[...])


_row_spec = pl.BlockSpec((_R, F), lambda i: (i, 0))
_mat_spec = pl.BlockSpec((F, F), lambda i: (0, 0))
_bias_spec = pl.BlockSpec((1, F), lambda i: (0, 0))
_degp_spec = pl.BlockSpec((NC, _R, DW), lambda i: (0, i, 0))
_half_spec = pl.BlockSpec((NC, _R, FH), lambda i: (0, i, 0))
_grid = (N // _R,)
_half_out = jax.ShapeDtypeStruct((NC, N, FH), jnp.float32)

_tc_in = pl.pallas_call(
    _tc_in_body, grid=_grid,
    in_specs=[_row_spec, _mat_spec, _bias_spec, _mat_spec, _degp_spec],
    out_specs=_half_spec, out_shape=_half_out)

_tc_mid = pl.pallas_call(
    _tc_mid_body, grid=_grid,
    in_specs=[_half_spec, _degp_spec, _bias_spec, _mat_spec],
    out_specs=_half_spec, out_shape=_half_out)

_tc_out = pl.pallas_call(
    _tc_out_body, grid=_grid,
    in_specs=[_half_spec, _degp_spec, _bias_spec],
    out_specs=_row_spec, out_shape=jax.ShapeDtypeStruct((N, F), jnp.float32))


def kernel(x, edge_index, edge_attr, W1, b1, We, be, Wc1, bc1, Wc2, bc2):
    del edge_attr, We, be
    src2 = edge_index[0].reshape(NS * KB, B)
    dst2 = edge_index[1].reshape(NS * KB, B)
    ones = jnp.ones((B, DW), jnp.float32)
    zeros = jnp.zeros((N, DW), jnp.float32)

    degp = _deg_sc(dst2, ones, zeros)
    g1 = _tc_in(x, W1, b1.reshape(1, F), Wc1, degp)
    s1 = _spmm_sc(g1, src2, dst2)
    g2 = _tc_mid(s1, degp, bc1.reshape(1, F), Wc2)
    s2 = _spmm_sc(g2, src2, dst2)
    return _tc_out(s2, degp, bc2.reshape(1, F))


# speedup vs baseline: 28.4416x; 1.0470x over previous
import functools

import jax
import jax.numpy as jnp
from jax import lax
from jax.experimental import pallas as pl
from jax.experimental.pallas import tpu as pltpu
from jax.experimental.pallas import tpu_sc as plsc

N = 10000
E = 320000
F = 128
FH = F // 2
NC = 2
NS = 16
B = 125
EPS = E // NS
KB = EPS // B
NBUF = 4
RPS = 624
TAIL = N - NS * RPS
DW = 16

_mesh = plsc.VectorSubcoreMesh(
    core_axis_name="c", subcore_axis_name="s", num_cores=NC, num_subcores=NS)


def _rows_copy(src_ref, dst_ref, sid):
    pltpu.sync_copy(src_ref.at[pl.ds(sid * RPS, RPS)],
                    dst_ref.at[pl.ds(sid * RPS, RPS)])

    @pl.when(sid == 0)
    def _():
        pltpu.sync_copy(src_ref.at[pl.ds(NS * RPS, TAIL)],
                        dst_ref.at[pl.ds(NS * RPS, TAIL)])


@functools.partial(
    pl.kernel,
    out_type=jax.ShapeDtypeStruct((NC, N, DW), jnp.float32),
    mesh=_mesh,
    compiler_params=pltpu.CompilerParams(use_tc_tiling_on_sc=False),
    scratch_types=[
        pltpu.VMEM((KB // 2, B), jnp.int32),
        pltpu.VMEM((B, DW), jnp.float32),
        pltpu.VMEM_SHARED((N, DW), jnp.float32),
    ],
)
def _deg_sc(eidx_hbm, ones_hbm, zeros_hbm, out_hbm, dst_v, ones_v, acc):
    cid = lax.axis_index("c")
    sid = lax.axis_index("s")
    wid = cid * NS + sid
    _rows_copy(zeros_hbm, acc, sid)
    pltpu.sync_copy(eidx_hbm.at[1].at[pl.ds(wid * (KB // 2), KB // 2)], dst_v)
    pltpu.sync_copy(ones_hbm, ones_v)
    plsc.subcore_barrier()

    def body(j, _):
        pltpu.sync_copy(ones_v, acc.at[dst_v.at[j]], add=True)
        return 0

    lax.fori_loop(0, KB // 2, body, 0)
    plsc.subcore_barrier()
    _rows_copy(acc, out_hbm.at[cid], sid)


@functools.partial(
    pl.kernel,
    out_type=jax.ShapeDtypeStruct((NC, N, FH), jnp.float32),
    mesh=_mesh,
    compiler_params=pltpu.CompilerParams(use_tc_tiling_on_sc=False),
    scratch_types=[
        pltpu.VMEM((KB, B), jnp.int32),
        pltpu.VMEM((KB, B), jnp.int32),
        pltpu.VMEM((NBUF, B, FH), jnp.float32),
        pltpu.VMEM_SHARED((N, FH), jnp.float32),
        pltpu.SemaphoreType.DMA((NBUF,)),
        pltpu.SemaphoreType.DMA((NBUF,)),
    ],
)
def _spmm_sc(g_hbm, eidx_hbm, out_hbm,
             src_v, dst_v, bufs, acc, gsem, ssem):
    cid = lax.axis_index("c")
    sid = lax.axis_index("s")
    gh = g_hbm.at[cid]
    _rows_copy(gh, acc, sid)
    pltpu.sync_copy(eidx_hbm.at[0].at[pl.ds(sid * KB, KB)], src_v)
    pltpu.sync_copy(eidx_hbm.at[1].at[pl.ds(sid * KB, KB)], dst_v)
    plsc.subcore_barrier()

    for b in range(NBUF):
        pltpu.async_copy(gh.at[src_v.at[b]], bufs.at[b], gsem.at[b])

    def body(i, _):
        for b in range(NBUF):
            j = i * NBUF + b
            pltpu.make_async_copy(
                gh.at[src_v.at[j]], bufs.at[b], gsem.at[b]).wait()
            pltpu.async_copy(bufs.at[b], acc.at[dst_v.at[j]], ssem.at[b],
                             add=True)
            c = (b + NBUF // 2) % NBUF
            k = j - NBUF // 2

            @pl.when(jnp.logical_and(k >= 0, k + NBUF < KB))
            def _():
                pltpu.make_async_copy(
                    bufs.at[c], acc.at[dst_v.at[k]], ssem.at[c]).wait()
                pltpu.async_copy(
                    gh.at[src_v.at[k + NBUF]], bufs.at[c], gsem.at[c])

        return 0

    lax.fori_loop(0, KB // NBUF, body, 0)
    for b in range(NBUF):
        pltpu.make_async_copy(
            bufs.at[b], acc.at[dst_v.at[KB - NBUF + b]], ssem.at[b]).wait()

    plsc.subcore_barrier()
    _rows_copy(acc, out_hbm.at[cid], sid)


_R = 2000


def _dinv_of(degp_blk):
    deg = degp_blk[0] + degp_blk[1]
    return lax.rsqrt(deg[:, :1] + 1.0)


def _halves(rows):
    return jnp.stack([rows[:, :FH], rows[:, FH:]], axis=0)


def _cat(sp_ref):
    return jnp.concatenate([sp_ref[0], sp_ref[1]], axis=1)


def _tc_in_body(x_ref, w1_ref, b1_ref, wc1_ref, degp_ref, g1_ref):
    h0 = jnp.dot(x_ref[...], w1_ref[...],
                 preferred_element_type=jnp.float32) + b1_ref[...]
    g1_ref[...] = _halves(_dinv_of(degp_ref) * jnp.dot(
        h0, wc1_ref[...], preferred_element_type=jnp.float32))


def _tc_mid_body(sp_ref, degp_ref, b_ref, w_ref, out_ref):
    dinv = _dinv_of(degp_ref)
    h = jax.nn.relu(dinv * _cat(sp_ref) + b_ref[...])
    out_ref[...] = _halves(
        dinv * jnp.dot(h, w_ref[...], preferred_element_type=jnp.float32))


def _tc_out_body(sp_ref, degp_ref, b_ref, out_ref):
    out_ref[...] = jax.nn.relu(
        _dinv_of(degp_ref) * _cat(sp_ref) + b_ref[...])


_row_spec = pl.BlockSpec((_R, F), lambda i: (i, 0))
_mat_spec = pl.BlockSpec((F, F), lambda i: (0, 0))
_bias_spec = pl.BlockSpec((1, F), lambda i: (0, 0))
_degp_spec = pl.BlockSpec((NC, _R, DW), lambda i: (0, i, 0))
_half_spec = pl.BlockSpec((NC, _R, FH), lambda i: (0, i, 0))
_grid = (N // _R,)
_half_out = jax.ShapeDtypeStruct((NC, N, FH), jnp.float32)

_tc_in = pl.pallas_call(
    _tc_in_body, grid=_grid,
    in_specs=[_row_spec, _mat_spec, _bias_spec, _mat_spec, _degp_spec],
    out_specs=_half_spec, out_shape=_half_out)

_tc_mid = pl.pallas_call(
    _tc_mid_body, grid=_grid,
    in_specs=[_half_spec, _degp_spec, _bias_spec, _mat_spec],
    out_specs=_half_spec, out_shape=_half_out)

_tc_out = pl.pallas_call(
    _tc_out_body, grid=_grid,
    in_specs=[_half_spec, _degp_spec, _bias_spec],
    out_specs=_row_spec, out_shape=jax.ShapeDtypeStruct((N, F), jnp.float32))


def kernel(x, edge_index, edge_attr, W1, b1, We, be, Wc1, bc1, Wc2, bc2):
    del edge_attr, We, be
    eidx = edge_index.reshape(2, NS * KB, B)
    ones = jnp.ones((B, DW), jnp.float32)
    zeros = jnp.zeros((N, DW), jnp.float32)

    degp = _deg_sc(eidx, ones, zeros)
    g1 = _tc_in(x, W1, b1.reshape(1, F), Wc1, degp)
    s1 = _spmm_sc(g1, eidx)
    g2 = _tc_mid(s1, degp, bc1.reshape(1, F), Wc2)
    s2 = _spmm_sc(g2, eidx)
    return _tc_out(s2, degp, bc2.reshape(1, F))


# speedup vs baseline: 28.4785x; 1.0013x over previous
import functools

import jax
import jax.numpy as jnp
from jax import lax
from jax.experimental import pallas as pl
from jax.experimental.pallas import tpu as pltpu
from jax.experimental.pallas import tpu_sc as plsc

N = 10000
E = 320000
F = 128
FH = F // 2
NC = 2
NS = 16
B = 125
EPS = E // NS
KB = EPS // B
NBUF = 4
RPS = 624
TAIL = N - NS * RPS
DW = 16

_mesh = plsc.VectorSubcoreMesh(
    core_axis_name="c", subcore_axis_name="s", num_cores=NC, num_subcores=NS)


def _rows_copy(src_ref, dst_ref, sid):
    pltpu.sync_copy(src_ref.at[pl.ds(sid * RPS, RPS)],
                    dst_ref.at[pl.ds(sid * RPS, RPS)])

    @pl.when(sid == 0)
    def _():
        pltpu.sync_copy(src_ref.at[pl.ds(NS * RPS, TAIL)],
                        dst_ref.at[pl.ds(NS * RPS, TAIL)])


@functools.partial(
    pl.kernel,
    out_type=jax.ShapeDtypeStruct((NC, N, DW), jnp.float32),
    mesh=_mesh,
    compiler_params=pltpu.CompilerParams(use_tc_tiling_on_sc=False),
    scratch_types=[
        pltpu.VMEM((KB // 2, B), jnp.int32),
        pltpu.VMEM((B, DW), jnp.float32),
        pltpu.VMEM_SHARED((N, DW), jnp.float32),
    ],
)
def _deg_sc(eidx_hbm, ones_hbm, zeros_hbm, out_hbm, dst_v, ones_v, acc):
    cid = lax.axis_index("c")
    sid = lax.axis_index("s")
    wid = cid * NS + sid
    _rows_copy(zeros_hbm, acc, sid)
    pltpu.sync_copy(eidx_hbm.at[1].at[pl.ds(wid * (KB // 2), KB // 2)], dst_v)
    pltpu.sync_copy(ones_hbm, ones_v)
    plsc.subcore_barrier()

    def body(j, _):
        pltpu.sync_copy(ones_v, acc.at[dst_v.at[j]], add=True)
        return 0

    lax.fori_loop(0, KB // 2, body, 0)
    plsc.subcore_barrier()
    _rows_copy(acc, out_hbm.at[cid], sid)


@functools.partial(
    pl.kernel,
    out_type=jax.ShapeDtypeStruct((NC, N, FH), jnp.float32),
    mesh=_mesh,
    compiler_params=pltpu.CompilerParams(use_tc_tiling_on_sc=False),
    scratch_types=[
        pltpu.VMEM((KB, B), jnp.int32),
        pltpu.VMEM((KB, B), jnp.int32),
        pltpu.VMEM((NBUF, B, FH), jnp.float32),
        pltpu.VMEM_SHARED((N, FH), jnp.float32),
        pltpu.SemaphoreType.DMA((NBUF,)),
        pltpu.SemaphoreType.DMA((NBUF,)),
    ],
)
def _spmm_sc(g_hbm, eidx_hbm, out_hbm,
             src_v, dst_v, bufs, acc, gsem, ssem):
    cid = lax.axis_index("c")
    sid = lax.axis_index("s")
    gh = g_hbm.at[cid]
    _rows_copy(gh, acc, sid)
    pltpu.sync_copy(eidx_hbm.at[0].at[pl.ds(sid * KB, KB)], src_v)
    pltpu.sync_copy(eidx_hbm.at[1].at[pl.ds(sid * KB, KB)], dst_v)
    plsc.subcore_barrier()

    for b in range(NBUF):
        pltpu.async_copy(gh.at[src_v.at[b]], bufs.at[b], gsem.at[b])

    def body(i, _):
        for b in range(NBUF):
            j = i * NBUF + b
            pltpu.make_async_copy(
                gh.at[src_v.at[j]], bufs.at[b], gsem.at[b]).wait()
            pltpu.async_copy(bufs.at[b], acc.at[dst_v.at[j]], ssem.at[b],
                             add=True)
            c = (b - NBUF // 2) % NBUF
            k = j - NBUF // 2

            @pl.when(jnp.logical_and(k >= 0, k + NBUF < KB))
            def _():
                pltpu.make_async_copy(
                    bufs.at[c], acc.at[dst_v.at[k]], ssem.at[c]).wait()
                pltpu.async_copy(
                    gh.at[src_v.at[k + NBUF]], bufs.at[c], gsem.at[c])

        return 0

    lax.fori_loop(0, KB // NBUF, body, 0)
    for b in range(NBUF):
        pltpu.make_async_copy(
            bufs.at[b], acc.at[dst_v.at[KB - NBUF + b]], ssem.at[b]).wait()

    plsc.subcore_barrier()
    _rows_copy(acc, out_hbm.at[cid], sid)


_R = 2000


def _dinv_of(degp_blk):
    deg = degp_blk[0] + degp_blk[1]
    return lax.rsqrt(deg[:, :1] + 1.0)


def _halves(rows):
    return jnp.stack([rows[:, :FH], rows[:, FH:]], axis=0)


def _cat(sp_ref):
    return jnp.concatenate([sp_ref[0], sp_ref[1]], axis=1)


def _tc_in_body(x_ref, w1_ref, b1_ref, wc1_ref, degp_ref, g1_ref):
    h0 = jnp.dot(x_ref[...], w1_ref[...],
                 preferred_element_type=jnp.float32) + b1_ref[...]
    g1_ref[...] = _halves(_dinv_of(degp_ref) * jnp.dot(
        h0, wc1_ref[...], preferred_element_type=jnp.float32))


def _tc_mid_body(sp_ref, degp_ref, b_ref, w_ref, out_ref):
    dinv = _dinv_of(degp_ref)
    h = jax.nn.relu(dinv * _cat(sp_ref) + b_ref[...])
    out_ref[...] = _halves(
        dinv * jnp.dot(h, w_ref[...], preferred_element_type=jnp.float32))


def _tc_out_body(sp_ref, degp_ref, b_ref, out_ref):
    out_ref[...] = jax.nn.relu(
        _dinv_of(degp_ref) * _cat(sp_ref) + b_ref[...])


_row_spec = pl.BlockSpec((_R, F), lambda i: (i, 0))
_mat_spec = pl.BlockSpec((F, F), lambda i: (0, 0))
_bias_spec = pl.BlockSpec((1, F), lambda i: (0, 0))
_degp_spec = pl.BlockSpec((NC, _R, DW), lambda i: (0, i, 0))
_half_spec = pl.BlockSpec((NC, _R, FH), lambda i: (0, i, 0))
_grid = (N // _R,)
_half_out = jax.ShapeDtypeStruct((NC, N, FH), jnp.float32)

_tc_in = pl.pallas_call(
    _tc_in_body, grid=_grid,
    in_specs=[_row_spec, _mat_spec, _bias_spec, _mat_spec, _degp_spec],
    out_specs=_half_spec, out_shape=_half_out)

_tc_mid = pl.pallas_call(
    _tc_mid_body, grid=_grid,
    in_specs=[_half_spec, _degp_spec, _bias_spec, _mat_spec],
    out_specs=_half_spec, out_shape=_half_out)

_tc_out = pl.pallas_call(
    _tc_out_body, grid=_grid,
    in_specs=[_half_spec, _degp_spec, _bias_spec],
    out_specs=_row_spec, out_shape=jax.ShapeDtypeStruct((N, F), jnp.float32))


def kernel(x, edge_index, edge_attr, W1, b1, We, be, Wc1, bc1, Wc2, bc2):
    del edge_attr, We, be
    eidx = edge_index.reshape(2, NS * KB, B)
    ones = jnp.ones((B, DW), jnp.float32)
    zeros = jnp.zeros((N, DW), jnp.float32)

    degp = _deg_sc(eidx, ones, zeros)
    g1 = _tc_in(x, W1, b1.reshape(1, F), Wc1, degp)
    s1 = _spmm_sc(g1, eidx)
    g2 = _tc_mid(s1, degp, bc1.reshape(1, F), Wc2)
    s2 = _spmm_sc(g2, eidx)
    return _tc_out(s2, degp, bc2.reshape(1, F))
